# baseline (speedup 1.0000x reference)
import jax
import jax.numpy as jnp
from jax.experimental import pallas as pl

N = 10000
E = 320000
D_IN = 128
D_H = 128
D_OUT = 2

_PREC = jax.lax.Precision.HIGHEST


def _leaky(x):
    return jnp.where(x >= 0.0, x, 0.01 * x)


def _stage1_body(a, cnt, w, b, out):
    h = a[:] / jnp.maximum(cnt[:], 1.0)
    mask = jnp.where(cnt[:] > 0.0, 1.0, 0.0)
    out[:] = _leaky(jnp.dot(h, w[:], precision=_PREC,
                            preferred_element_type=jnp.float32) + b[:] * mask)


def _stage1(a, cnt, w, b):
    return pl.pallas_call(
        _stage1_body,
        out_shape=jax.ShapeDtypeStruct((N, D_H), jnp.float32),
    )(a, cnt, w, b)


def _stage2_body(bsum, cnt, w, b, wo, bo, out):
    h = bsum[:] / jnp.maximum(cnt[:], 1.0)
    mask = jnp.where(cnt[:] > 0.0, 1.0, 0.0)
    h2 = jnp.dot(h, w[:], precision=_PREC,
                 preferred_element_type=jnp.float32) + b[:] * mask
    out[:] = jnp.dot(h2, wo[:], precision=_PREC,
                     preferred_element_type=jnp.float32) + bo[:]


def _stage2(bsum, cnt, w, b, wo, bo):
    return pl.pallas_call(
        _stage2_body,
        out_shape=jax.ShapeDtypeStruct((N, D_H), jnp.float32),
    )(bsum, cnt, w, b, wo, bo)


def kernel(features, edge_index_t2e, edge_index_e2t, embed_entity,
           W_t2e_0, b_t2e_0, W_e2t_0, b_e2t_0,
           W_t2e_1, b_t2e_1, W_e2t_1, b_e2t_1,
           W_out, b_out):
    src_te = edge_index_t2e[0]
    dst_te = edge_index_t2e[1]
    src_et = edge_index_e2t[0]
    dst_et = edge_index_e2t[1]

    a = jax.ops.segment_sum(jnp.take(features, src_te, axis=0), dst_te,
                            num_segments=N)
    cnt_e = jax.ops.segment_sum(jnp.ones((E,), jnp.float32), dst_te,
                                num_segments=N)
    cnt_e_b = jnp.broadcast_to(cnt_e[:, None], (N, D_H))

    g_entity = _stage1(a, cnt_e_b, W_t2e_0, b_t2e_0)

    bsum = jax.ops.segment_sum(jnp.take(g_entity, src_et, axis=0), dst_et,
                               num_segments=N)
    cnt_t = jax.ops.segment_sum(jnp.ones((E,), jnp.float32), dst_et,
                                num_segments=N)
    cnt_t_b = jnp.broadcast_to(cnt_t[:, None], (N, D_H))

    wo_p = jnp.pad(W_out, ((0, 0), (0, D_H - D_OUT)))
    bo_p = jnp.pad(b_out, (0, D_H - D_OUT))
    out = _stage2(bsum, cnt_t_b, W_e2t_1, b_e2t_1, wo_p, bo_p)
    return out[:, :D_OUT]
